# Initial kernel scaffold; baseline (speedup 1.0000x reference)
import jax
import jax.numpy as jnp
from jax.experimental import pallas as pl


def kernel(school_idx, goal_idx, method_idx, subject_feats, grade_feats, school_table, goal_table, method_table, W_subj, b_subj, W_grade, b_grade, W1, b1, W2, b2, W3, b3):
    raise NotImplementedError("write your pallas kernel here")


# speedup vs baseline: 1.6351x; 1.6351x over previous
import functools

import jax
import jax.numpy as jnp
from jax import lax
from jax.experimental import pallas as pl
from jax.experimental.pallas import tpu as pltpu
from jax.experimental.pallas import tpu_sc as plsc

B = 16384
D_EMB = 32

_NC = 2
_NS = 16
_NW = _NC * _NS
_BPW = B // _NW
_CHUNK = 128
_NCH = _BPW // _CHUNK

_TC_BLK = 2048


def _sc_gather_body(sidx, gidx, midx, stab, gtab, mtab,
                    out_s, out_g, out_m, idx_v, rows_v, sem):
    wid = lax.axis_index("s") * _NC + lax.axis_index("c")
    base = wid * _BPW
    idxs = (sidx, gidx, midx)
    tabs = (stab, gtab, mtab)
    outs = (out_s, out_g, out_m)
    for t in range(3):
        for j in range(_NCH):
            pltpu.sync_copy(idxs[t].at[pl.ds(base + j * _CHUNK, _CHUNK)],
                            idx_v.at[t, j])
    copies = []
    for t in range(3):
        for j in range(_NCH):
            copies.append(
                pltpu.async_copy(tabs[t].at[idx_v.at[t, j]],
                                 rows_v.at[t, j], sem))
    for c in copies:
        c.wait()
    for t in range(3):
        for j in range(_NCH):
            pltpu.sync_copy(rows_v.at[t, j],
                            outs[t].at[pl.ds(base + j * _CHUNK, _CHUNK)])


@jax.jit
def _sc_gather(school_idx, goal_idx, method_idx,
               school_table, goal_table, method_table):
    mesh = plsc.VectorSubcoreMesh(core_axis_name="c", subcore_axis_name="s")
    emb = jax.ShapeDtypeStruct((B, D_EMB), jnp.float32)
    return pl.kernel(
        _sc_gather_body,
        out_type=(emb, emb, emb),
        mesh=mesh,
        scratch_types=[
            pltpu.VMEM((3, _NCH, _CHUNK), jnp.int32),
            pltpu.VMEM((3, _NCH, _CHUNK, D_EMB), jnp.float32),
            pltpu.SemaphoreType.DMA,
        ],
        compiler_params=pltpu.CompilerParams(use_tc_tiling_on_sc=False),
    )(school_idx, goal_idx, method_idx, school_table, goal_table,
      method_table)


def _tc_mlp_body(es, eg, em, sf, gf, wsub, bsub, wgrd, bgrd,
                 w1, b1, w2, b2, w3, b3, out):
    f32 = jnp.float32
    dot = functools.partial(jnp.dot, preferred_element_type=f32)
    w1_all = w1[...]
    subj = dot(sf[...], wsub[...]) + bsub[...]
    grd = dot(gf[...], wgrd[...]) + bgrd[...]
    x = (dot(es[...], w1_all[0:32])
         + dot(eg[...], w1_all[32:64])
         + dot(em[...], w1_all[64:96])
         + dot(subj, w1_all[96:128])
         + dot(grd, w1_all[128:160])
         + b1[...])
    h = jnp.maximum(x, 0.0)
    h = jnp.maximum(dot(h, w2[...]) + b2[...], 0.0)
    out[...] = dot(h, w3[...]) + b3[...]


@jax.jit
def _tc_mlp(e_s, e_g, e_m, subject_feats, grade_feats,
            W_subj, b_subj, W_grade, b_grade, W1, b1, W2, b2, W3, b3):
    nblk = B // _TC_BLK
    row = lambda i: (i, 0)
    rep = lambda i: (0, 0)

    def spec(shape, index_map):
        return pl.BlockSpec(shape, index_map)

    return pl.pallas_call(
        _tc_mlp_body,
        grid=(nblk,),
        in_specs=[
            spec((_TC_BLK, 32), row),
            spec((_TC_BLK, 32), row),
            spec((_TC_BLK, 32), row),
            spec((_TC_BLK, 10), row),
            spec((_TC_BLK, 12), row),
            spec((10, 32), rep),
            spec((1, 32), rep),
            spec((12, 32), rep),
            spec((1, 32), rep),
            spec((160, 128), rep),
            spec((1, 128), rep),
            spec((128, 64), rep),
            spec((1, 64), rep),
            spec((64, 32), rep),
            spec((1, 32), rep),
        ],
        out_specs=spec((_TC_BLK, 32), row),
        out_shape=jax.ShapeDtypeStruct((B, 32), jnp.float32),
    )(e_s, e_g, e_m, subject_feats, grade_feats,
      W_subj, b_subj, W_grade, b_grade, W1, b1, W2, b2, W3, b3)


def kernel(school_idx, goal_idx, method_idx, subject_feats, grade_feats,
           school_table, goal_table, method_table,
           W_subj, b_subj, W_grade, b_grade, W1, b1, W2, b2, W3, b3):
    e_s, e_g, e_m = _sc_gather(school_idx, goal_idx, method_idx,
                               school_table, goal_table, method_table)
    return _tc_mlp(e_s, e_g, e_m, subject_feats, grade_feats,
                   W_subj, b_subj.reshape(1, -1), W_grade,
                   b_grade.reshape(1, -1), W1, b1.reshape(1, -1),
                   W2, b2.reshape(1, -1), W3, b3.reshape(1, -1))
